# baseline (speedup 1.0000x reference)
import functools

import jax
import jax.numpy as jnp
from jax import lax
from jax.experimental import pallas as pl
from jax.experimental.pallas import tpu as pltpu
from jax.experimental.pallas import tpu_sc as plsc

NB = 128
NP = 8192
G = 64
NC, NS, L = 2, 16, 16
NW = NC * NS
BPW = NB // NW
PCELLS = G * 8 * 1024
ZCHUNK = 32768
NZ = PCELLS // ZCHUNK
ROWS = NP // 128
_ABL = 2


def _sc_body(coords_hbm, marks_hbm, zeros_v, coords_v, idx0_v, idx1_v,
             idx2_v, idx3_v, ones_v, zsem, ssem):
    idx_bufs = [idx0_v, idx1_v, idx2_v, idx3_v]
    wid = lax.axis_index("s") * NC + lax.axis_index("c")
    lanes = lax.iota(jnp.int32, L)

    def fill_ones(i, _):
        ones_v[pl.ds(i * L, L)] = jnp.full((L,), 1.0, jnp.float32)
        return 0
    lax.fori_loop(0, NP // L, fill_ones, 0)

    def fill_zero(i, _):
        zeros_v[pl.ds(i * L, L)] = jnp.zeros((L,), jnp.float32)
        return 0
    lax.fori_loop(0, ZCHUNK // L, fill_zero, 0)

    zcopies = [
        pltpu.async_copy(
            zeros_v,
            marks_hbm.at[pl.ds((wid * BPW + bl) * PCELLS + z * ZCHUNK,
                               ZCHUNK)],
            zsem)
        for bl in range(BPW)
        for z in range(NZ)
    ]

    for bl in range(BPW if _ABL >= 1 else 0):
        b = wid * BPW + bl
        base = b * PCELLS
        idx_v = idx_bufs[bl]
        pltpu.sync_copy(coords_hbm.at[pl.ds(b * NP * 3, NP * 3)], coords_v)

        def idx_row(j, _):
            for k in range(8):
                p3 = (j * 128 + k * L + lanes) * 3
                x = plsc.load_gather(coords_v, [p3])
                y = plsc.load_gather(coords_v, [p3 + 1])
                z = plsc.load_gather(coords_v, [p3 + 2])
                ix = (x * 64.0).astype(jnp.int32)
                iy = (y * 64.0).astype(jnp.int32)
                iz = (z * 64.0).astype(jnp.int32)
                cell = ((ix * 8 + (iz >> 3)) * 8 + (iz & 7)) * 128 + iy
                idx_v[pl.ds(j * 128 + k * L, L)] = base + cell
            return 0
        lax.fori_loop(0, ROWS, idx_row, 0)

    for c in zcopies:
        c.wait()
    scopies = [
        pltpu.async_copy(ones_v, marks_hbm.at[idx_bufs[bl]], ssem)
        for bl in range(BPW if _ABL >= 2 else 0)
    ]
    for c in scopies:
        c.wait()


_mesh = plsc.VectorSubcoreMesh(core_axis_name="c", subcore_axis_name="s")

_scatter = functools.partial(
    pl.kernel,
    out_type=jax.ShapeDtypeStruct((NB * PCELLS,), jnp.float32),
    mesh=_mesh,
    scratch_types=[
        pltpu.VMEM((ZCHUNK,), jnp.float32),
        pltpu.VMEM((NP * 3,), jnp.float32),
        pltpu.VMEM((NP,), jnp.int32),
        pltpu.VMEM((NP,), jnp.int32),
        pltpu.VMEM((NP,), jnp.int32),
        pltpu.VMEM((NP,), jnp.int32),
        pltpu.VMEM((NP,), jnp.float32),
        pltpu.SemaphoreType.DMA,
        pltpu.SemaphoreType.DMA,
    ],
    compiler_params=pltpu.CompilerParams(needs_layout_passes=False),
)(_sc_body)


def _expand_body(m_ref, o_ref):
    for i1 in range(G):
        o_ref[0, i1] = m_ref[pl.ds(i1 * G, G), :G]


_expand = pl.pallas_call(
    _expand_body,
    grid=(NB,),
    in_specs=[pl.BlockSpec((PCELLS // 128, 128), lambda b: (b, 0))],
    out_specs=pl.BlockSpec((1, G, G, G), lambda b: (b, 0, 0, 0)),
    out_shape=jax.ShapeDtypeStruct((NB, G, G, G), jnp.float32),
)


def kernel(coord_v):
    marks = _scatter(coord_v.reshape(NB * NP * 3))
    return _expand(marks.reshape(NB * PCELLS // 128, 128))


# speedup vs baseline: 1.0003x; 1.0003x over previous
import functools

import jax
import jax.numpy as jnp
from jax import lax
from jax.experimental import pallas as pl
from jax.experimental.pallas import tpu as pltpu
from jax.experimental.pallas import tpu_sc as plsc

NB = 128
NP = 8192
G = 64
NC, NS, L = 2, 16, 16
NW = NC * NS
BPW = NB // NW
PCELLS = G * 8 * 1024
ZCHUNK = 32768
NZ = PCELLS // ZCHUNK
ROWS = NP // 128
_ABL = 2


def _sc_body(coords_hbm, marks_hbm, zeros_v, coords_v, idx0_v, idx1_v,
             idx2_v, idx3_v, ones_v, zsem, ssem):
    idx_bufs = [idx0_v, idx1_v, idx2_v, idx3_v]
    wid = lax.axis_index("s") * NC + lax.axis_index("c")
    lanes = lax.iota(jnp.int32, L)

    def fill_ones(i, _):
        ones_v[pl.ds(i * L, L)] = jnp.full((L,), 1.0, jnp.float32)
        return 0
    lax.fori_loop(0, NP // L, fill_ones, 0)

    def fill_zero(i, _):
        zeros_v[pl.ds(i * L, L)] = jnp.zeros((L,), jnp.float32)
        return 0
    lax.fori_loop(0, ZCHUNK // L, fill_zero, 0)

    zcopies = [
        pltpu.async_copy(
            zeros_v,
            marks_hbm.at[pl.ds((wid * BPW + bl) * PCELLS + z * ZCHUNK,
                               ZCHUNK)],
            zsem)
        for bl in range(BPW)
        for z in range(NZ)
    ]

    for bl in range(BPW if _ABL >= 1 else 0):
        b = wid * BPW + bl
        base = b * PCELLS
        idx_v = idx_bufs[bl]
        pltpu.sync_copy(coords_hbm.at[pl.ds(b * NP * 3, NP * 3)], coords_v)

        def idx_row(j, _):
            for k in range(8):
                p3 = (j * 128 + k * L + lanes) * 3
                x = plsc.load_gather(coords_v, [p3])
                y = plsc.load_gather(coords_v, [p3 + 1])
                z = plsc.load_gather(coords_v, [p3 + 2])
                ix = (x * 64.0).astype(jnp.int32)
                iy = (y * 64.0).astype(jnp.int32)
                iz = (z * 64.0).astype(jnp.int32)
                cell = ((ix * 8 + (iz >> 3)) * 8 + (iz & 7)) * 128 + iy
                idx_v[pl.ds(j * 128 + k * L, L)] = base + cell
            return 0
        lax.fori_loop(0, ROWS, idx_row, 0)

    for c in zcopies:
        c.wait()
    scopies = [
        pltpu.async_copy(ones_v, marks_hbm.at[idx_bufs[bl]], ssem)
        for bl in range(BPW if _ABL >= 2 else 0)
    ]
    for c in scopies:
        c.wait()


_mesh = plsc.VectorSubcoreMesh(core_axis_name="c", subcore_axis_name="s")

_scatter = functools.partial(
    pl.kernel,
    out_type=jax.ShapeDtypeStruct((NB * PCELLS,), jnp.float32),
    mesh=_mesh,
    scratch_types=[
        pltpu.VMEM((ZCHUNK,), jnp.float32),
        pltpu.VMEM((NP * 3,), jnp.float32),
        pltpu.VMEM((NP,), jnp.int32),
        pltpu.VMEM((NP,), jnp.int32),
        pltpu.VMEM((NP,), jnp.int32),
        pltpu.VMEM((NP,), jnp.int32),
        pltpu.VMEM((NP,), jnp.float32),
        pltpu.SemaphoreType.DMA,
        pltpu.SemaphoreType.DMA,
    ],
    compiler_params=pltpu.CompilerParams(needs_layout_passes=False),
)(_sc_body)


def _expand_body(m_ref, o_ref):
    for i1 in range(G):
        v = m_ref[pl.ds(i1 * G * 128, G * 128)].reshape(G, 128)
        o_ref[0, i1] = v[:, :G]


_expand = pl.pallas_call(
    _expand_body,
    grid=(NB,),
    in_specs=[pl.BlockSpec((PCELLS,), lambda b: (b,))],
    out_specs=pl.BlockSpec((1, G, G, G), lambda b: (b, 0, 0, 0)),
    out_shape=jax.ShapeDtypeStruct((NB, G, G, G), jnp.float32),
)


def kernel(coord_v):
    marks = _scatter(coord_v.reshape(NB * NP * 3))
    return _expand(marks)


# speedup vs baseline: 4.4741x; 4.4726x over previous
import functools

import jax
import jax.numpy as jnp
from jax import lax
from jax.experimental import pallas as pl
from jax.experimental.pallas import tpu as pltpu
from jax.experimental.pallas import tpu_sc as plsc

NB = 128
NP = 8192
G = 64
NC, NS, L = 2, 16, 16
NW = NC * NS
BPW = NB // NW
PLANE = NB * NP
NCELL = G * G * G * NB


def _sc_body(coords_hbm, marks_ref, xs_v, ys_v, zs_v, idx0_v, idx1_v,
             idx2_v, idx3_v, ones_v, ssem):
    idx_bufs = [idx0_v, idx1_v, idx2_v, idx3_v]
    wid = lax.axis_index("s") * NC + lax.axis_index("c")

    def fill_ones(i, _):
        ones_v[pl.ds(i * L, L)] = jnp.full((L,), 1.0, jnp.float32)
        return 0
    lax.fori_loop(0, NP // L, fill_ones, 0)

    scopies = []
    for bl in range(BPW):
        b = wid * BPW + bl
        idx_v = idx_bufs[bl]
        pltpu.sync_copy(coords_hbm.at[pl.ds(b * NP, NP)], xs_v)
        pltpu.sync_copy(coords_hbm.at[pl.ds(PLANE + b * NP, NP)], ys_v)
        pltpu.sync_copy(coords_hbm.at[pl.ds(2 * PLANE + b * NP, NP)], zs_v)

        def idx_chunk(i, _):
            s = pl.ds(i * L, L)
            ix = (xs_v[s] * 64.0).astype(jnp.int32)
            iy = (ys_v[s] * 64.0).astype(jnp.int32)
            iz = (zs_v[s] * 64.0).astype(jnp.int32)
            idx_v[s] = ((ix * G + iz) * G + iy) * 128 + b
            return 0
        lax.fori_loop(0, NP // L, idx_chunk, 0)
        scopies.append(
            pltpu.async_copy(ones_v, marks_ref.at[idx_v], ssem))

    for c in scopies:
        c.wait()


_mesh = plsc.VectorSubcoreMesh(core_axis_name="c", subcore_axis_name="s")

_scatter = functools.partial(
    pl.kernel,
    out_type=(),
    mesh=_mesh,
    scratch_types=[
        pltpu.VMEM((NP,), jnp.float32),
        pltpu.VMEM((NP,), jnp.float32),
        pltpu.VMEM((NP,), jnp.float32),
        pltpu.VMEM((NP,), jnp.int32),
        pltpu.VMEM((NP,), jnp.int32),
        pltpu.VMEM((NP,), jnp.int32),
        pltpu.VMEM((NP,), jnp.int32),
        pltpu.VMEM((NP,), jnp.float32),
        pltpu.SemaphoreType.DMA,
    ],
    compiler_params=pltpu.CompilerParams(needs_layout_passes=False),
)(_sc_body)


def kernel(coord_v):
    flat = coord_v.transpose(2, 0, 1).reshape(3 * PLANE)
    marks_ref = jax.new_ref(jnp.zeros((NCELL,), jnp.float32))
    _scatter(flat, marks_ref)
    marks = marks_ref[...]
    return marks.reshape(G, G, G, NB).transpose(3, 0, 1, 2)
